# baseline (speedup 1.0000x reference)
import functools

import jax
import jax.numpy as jnp
from jax import lax
from jax.experimental import pallas as pl
from jax.experimental.pallas import tpu as pltpu
from jax.experimental.pallas import tpu_sc as plsc

T = 2048
D = 1024
E = 8
F = 2048
C = 512
DUMMY = E * C
NW = 32
TPW = T // NW
LANES = 16


def _gating_body(x_ref, wg_ref, sd1_ref, sd2_ref, sc1_ref, sc2_ref,
                 we1_ref, we2_ref, laux_ref, counts_ref):
    x = x_ref[...]
    wg = wg_ref[...]
    logits = jnp.dot(x, wg, preferred_element_type=jnp.float32)
    lmax = jnp.max(logits, axis=-1, keepdims=True)
    ex = jnp.exp(logits - lmax)
    gates = ex / jnp.sum(ex, axis=-1, keepdims=True)

    col = lax.broadcasted_iota(jnp.int32, (T, E), 1)
    gmax = jnp.max(gates, axis=-1, keepdims=True)
    idx1 = jnp.min(jnp.where(gates == gmax, col, E), axis=-1, keepdims=True)
    mask1 = col == idx1
    neg_inf = jnp.float32(-jnp.inf)
    logits2 = jnp.where(mask1, neg_inf, logits)
    l2max = jnp.max(logits2, axis=-1, keepdims=True)
    idx2 = jnp.min(jnp.where(logits2 == l2max, col, E), axis=-1, keepdims=True)
    mask2 = col == idx2

    m1f = mask1.astype(jnp.float32)
    m2f = mask2.astype(jnp.float32)

    me = jnp.mean(gates, axis=0, keepdims=True)
    ce = jnp.mean(m1f, axis=0, keepdims=True)
    laux_ref[...] = jnp.sum(me * ce, axis=1, keepdims=True) * jnp.float32(E)

    mm = jnp.concatenate([m1f, m2f], axis=1)
    chunk = 512
    parts = []
    for c0 in range(0, T, chunk):
        ri = lax.broadcasted_iota(jnp.int32, (chunk, T), 0) + c0
        ci = lax.broadcasted_iota(jnp.int32, (chunk, T), 1)
        tri = (ci <= ri).astype(jnp.float32)
        parts.append(jnp.dot(tri, mm, preferred_element_type=jnp.float32))
    cs = jnp.concatenate(parts, axis=0)

    count1 = jnp.sum(m1f, axis=0, keepdims=True)
    loc1 = cs[:, :E] - 1.0
    loc2 = cs[:, E:] - 1.0 + count1

    cap = jnp.float32(C)
    m1c = m1f * (loc1 < cap).astype(jnp.float32)
    m2c = m2f * (loc2 < cap).astype(jnp.float32)

    counts_ref[0, :] = jnp.sum(m1c + m2c, axis=0).astype(jnp.int32)

    loc1s = jnp.sum(loc1 * m1c, axis=1, keepdims=True)
    loc2s = jnp.sum(loc2 * m2c, axis=1, keepdims=True)
    kept1 = jnp.sum(m1c, axis=1, keepdims=True)
    kept2 = jnp.sum(m2c, axis=1, keepdims=True)
    g1s = jnp.sum(gates * m1c, axis=1, keepdims=True)
    g2s = jnp.sum(gates * m2c, axis=1, keepdims=True)
    denom = jnp.maximum(g1s + g2s, 1e-9)
    we1_ref[...] = jnp.broadcast_to(g1s / denom, (T, LANES))
    we2_ref[...] = jnp.broadcast_to(g2s / denom, (T, LANES))

    base1 = idx1 * C + loc1s.astype(jnp.int32)
    base2 = idx2 * C + loc2s.astype(jnp.int32)
    sc1_ref[...] = base1
    sc2_ref[...] = base2
    sd1_ref[...] = jnp.where(kept1 > 0.5, base1, DUMMY)
    sd2_ref[...] = jnp.where(kept2 > 0.5, base2, DUMMY)


def _gating(x, wg):
    out_shapes = (
        jax.ShapeDtypeStruct((T, 1), jnp.int32),
        jax.ShapeDtypeStruct((T, 1), jnp.int32),
        jax.ShapeDtypeStruct((T, 1), jnp.int32),
        jax.ShapeDtypeStruct((T, 1), jnp.int32),
        jax.ShapeDtypeStruct((T, LANES), jnp.float32),
        jax.ShapeDtypeStruct((T, LANES), jnp.float32),
        jax.ShapeDtypeStruct((1, 1), jnp.float32),
        jax.ShapeDtypeStruct((1, E), jnp.int32),
    )
    return pl.pallas_call(_gating_body, out_shape=out_shapes)(x, wg)


def _dispatch_body(x_hbm, s1_hbm, s2_hbm, out_hbm, i1_v, i2_v, rows_v, sem):
    wid = lax.axis_index("s") * 2 + lax.axis_index("c")
    base = wid * TPW
    pltpu.sync_copy(s1_hbm.at[pl.ds(base, TPW)], i1_v)
    pltpu.sync_copy(s2_hbm.at[pl.ds(base, TPW)], i2_v)
    pltpu.sync_copy(x_hbm.at[pl.ds(base, TPW)], rows_v)
    pltpu.async_copy(rows_v, out_hbm.at[i1_v], sem).wait()
    pltpu.async_copy(rows_v, out_hbm.at[i2_v], sem).wait()


def _sc_mesh():
    return plsc.VectorSubcoreMesh(
        core_axis_name="c", subcore_axis_name="s", num_cores=2, num_subcores=16)


@functools.lru_cache(maxsize=None)
def _make_dispatch():
    return pl.kernel(
        _dispatch_body,
        out_type=jax.ShapeDtypeStruct(((E + 1) * C, D), jnp.float32),
        mesh=_sc_mesh(),
        scratch_types=[
            pltpu.VMEM((TPW,), jnp.int32),
            pltpu.VMEM((TPW,), jnp.int32),
            pltpu.VMEM((TPW, D), jnp.float32),
            pltpu.SemaphoreType.DMA,
        ],
    )


FB = 512


def _ffn_body(x_ref, w1_ref, b1_ref, w2_ref, b2_ref, out_ref):
    f = pl.program_id(1)
    w1b = w1_ref[0].astype(jnp.bfloat16)
    h = jnp.dot(x_ref[...].astype(jnp.bfloat16), w1b,
                preferred_element_type=jnp.float32)
    h = jnp.maximum(h + b1_ref[0], 0.0)
    w2b = w2_ref[0].astype(jnp.bfloat16)
    contrib = jnp.dot(h.astype(jnp.bfloat16), w2b,
                      preferred_element_type=jnp.float32)

    @pl.when(f == 0)
    def _init():
        out_ref[...] = contrib + b2_ref[0]

    @pl.when(f != 0)
    def _acc():
        out_ref[...] += contrib


def _ffn(disp, w1, b1, w2, b2):
    grid = (E, F // FB)
    return pl.pallas_call(
        _ffn_body,
        grid=grid,
        in_specs=[
            pl.BlockSpec((C, D), lambda e, f: (e, 0)),
            pl.BlockSpec((1, D, FB), lambda e, f: (e, 0, f)),
            pl.BlockSpec((1, 1, FB), lambda e, f: (e, 0, f)),
            pl.BlockSpec((1, FB, D), lambda e, f: (e, f, 0)),
            pl.BlockSpec((1, 1, D), lambda e, f: (e, 0, 0)),
        ],
        out_specs=pl.BlockSpec((C, D), lambda e, f: (e, 0)),
        out_shape=jax.ShapeDtypeStruct((E * C, D), jnp.float32),
    )(disp, w1, b1.reshape(E, 1, F), w2, b2.reshape(E, 1, D))


CHUNK = 32


def _combine_body(eo_hbm, s1_hbm, s2_hbm, w1_hbm, w2_hbm, out_hbm,
                  i_v, r1_v, r2_v, w1_v, w2_v, sem):
    wid = lax.axis_index("s") * 2 + lax.axis_index("c")
    base = wid * TPW
    for half in range(TPW // CHUNK):
        b = base + half * CHUNK
        pltpu.sync_copy(w1_hbm.at[pl.ds(b, CHUNK)], w1_v)
        pltpu.sync_copy(w2_hbm.at[pl.ds(b, CHUNK)], w2_v)
        pltpu.sync_copy(s1_hbm.at[pl.ds(b, CHUNK)], i_v)
        c1 = pltpu.async_copy(eo_hbm.at[i_v], r1_v, sem)
        c1.wait()
        pltpu.sync_copy(s2_hbm.at[pl.ds(b, CHUNK)], i_v)
        c2 = pltpu.async_copy(eo_hbm.at[i_v], r2_v, sem)
        c2.wait()

        def token_body(t, carry):
            wa = w1_v[t, :]
            wb = w2_v[t, :]
            for j in range(D // LANES):
                sl = pl.ds(j * LANES, LANES)
                r1_v[t, sl] = r1_v[t, sl] * wa + r2_v[t, sl] * wb
            return carry

        lax.fori_loop(0, CHUNK, token_body, 0)
        pltpu.sync_copy(r1_v, out_hbm.at[pl.ds(b, CHUNK)])


@functools.lru_cache(maxsize=None)
def _make_combine():
    return pl.kernel(
        _combine_body,
        out_type=jax.ShapeDtypeStruct((T, D), jnp.float32),
        mesh=_sc_mesh(),
        scratch_types=[
            pltpu.VMEM((CHUNK,), jnp.int32),
            pltpu.VMEM((CHUNK, D), jnp.float32),
            pltpu.VMEM((CHUNK, D), jnp.float32),
            pltpu.VMEM((CHUNK, LANES), jnp.float32),
            pltpu.VMEM((CHUNK, LANES), jnp.float32),
            pltpu.SemaphoreType.DMA,
        ],
    )


def kernel(hidden_states, wg, w1, b1, w2, b2):
    S, B, _ = hidden_states.shape
    x = hidden_states.reshape(T, D)
    sd1, sd2, sc1, sc2, we1, we2, laux, counts = _gating(x, wg)
    disp = _make_dispatch()(x, sd1.reshape(T), sd2.reshape(T))
    eo = _ffn(disp, w1, b1, w2, b2)
    out = _make_combine()(eo, sc1.reshape(T), sc2.reshape(T), we1, we2)
    return (out.reshape(S, B, D), laux.reshape(()), counts.reshape(E))


# speedup vs baseline: 1.0554x; 1.0554x over previous
import functools

import jax
import jax.numpy as jnp
from jax import lax
from jax.experimental import pallas as pl
from jax.experimental.pallas import tpu as pltpu
from jax.experimental.pallas import tpu_sc as plsc

T = 2048
D = 1024
E = 8
F = 2048
C = 512
DUMMY = E * C
NW = 32
TPW = T // NW
LANES = 16


def _gating_body(x3_ref, wg_ref, sd1_ref, sd2_ref, sc1_ref, sc2_ref,
                 we1_ref, we2_ref, laux_ref, counts_ref):
    x = x3_ref[...].reshape(T, D)
    wg = wg_ref[...]
    logits = jnp.dot(x, wg, preferred_element_type=jnp.float32)
    lmax = jnp.max(logits, axis=-1, keepdims=True)
    ex = jnp.exp(logits - lmax)
    gates = ex / jnp.sum(ex, axis=-1, keepdims=True)

    col = lax.broadcasted_iota(jnp.int32, (T, E), 1)
    gmax = jnp.max(gates, axis=-1, keepdims=True)
    idx1 = jnp.min(jnp.where(gates == gmax, col, E), axis=-1, keepdims=True)
    mask1 = col == idx1
    neg_inf = jnp.float32(-jnp.inf)
    logits2 = jnp.where(mask1, neg_inf, logits)
    l2max = jnp.max(logits2, axis=-1, keepdims=True)
    idx2 = jnp.min(jnp.where(logits2 == l2max, col, E), axis=-1, keepdims=True)
    mask2 = col == idx2

    m1f = mask1.astype(jnp.float32)
    m2f = mask2.astype(jnp.float32)

    me = jnp.mean(gates, axis=0, keepdims=True)
    ce = jnp.mean(m1f, axis=0, keepdims=True)
    laux_ref[...] = jnp.sum(me * ce, axis=1, keepdims=True) * jnp.float32(E)

    mm = jnp.concatenate([m1f, m2f], axis=1)
    chunk = 512
    parts = []
    for c0 in range(0, T, chunk):
        ri = lax.broadcasted_iota(jnp.int32, (chunk, T), 0) + c0
        ci = lax.broadcasted_iota(jnp.int32, (chunk, T), 1)
        tri = (ci <= ri).astype(jnp.float32)
        parts.append(jnp.dot(tri, mm, preferred_element_type=jnp.float32))
    cs = jnp.concatenate(parts, axis=0)

    count1 = jnp.sum(m1f, axis=0, keepdims=True)
    loc1 = cs[:, :E] - 1.0
    loc2 = cs[:, E:] - 1.0 + count1

    cap = jnp.float32(C)
    m1c = m1f * (loc1 < cap).astype(jnp.float32)
    m2c = m2f * (loc2 < cap).astype(jnp.float32)

    counts_ref[0, :] = jnp.sum(m1c + m2c, axis=0).astype(jnp.int32)

    loc1s = jnp.sum(loc1 * m1c, axis=1, keepdims=True)
    loc2s = jnp.sum(loc2 * m2c, axis=1, keepdims=True)
    kept1 = jnp.sum(m1c, axis=1, keepdims=True)
    kept2 = jnp.sum(m2c, axis=1, keepdims=True)
    g1s = jnp.sum(gates * m1c, axis=1, keepdims=True)
    g2s = jnp.sum(gates * m2c, axis=1, keepdims=True)
    denom = jnp.maximum(g1s + g2s, 1e-9)
    we1_ref[...] = jnp.broadcast_to(g1s / denom, (T, LANES))
    we2_ref[...] = jnp.broadcast_to(g2s / denom, (T, LANES))

    base1 = idx1 * C + loc1s.astype(jnp.int32)
    base2 = idx2 * C + loc2s.astype(jnp.int32)
    sc1_ref[...] = base1.reshape(T)
    sc2_ref[...] = base2.reshape(T)
    sd1_ref[...] = jnp.where(kept1 > 0.5, base1, DUMMY).reshape(T)
    sd2_ref[...] = jnp.where(kept2 > 0.5, base2, DUMMY).reshape(T)


def _gating(x3, wg):
    out_shapes = (
        jax.ShapeDtypeStruct((T,), jnp.int32),
        jax.ShapeDtypeStruct((T,), jnp.int32),
        jax.ShapeDtypeStruct((T,), jnp.int32),
        jax.ShapeDtypeStruct((T,), jnp.int32),
        jax.ShapeDtypeStruct((T, LANES), jnp.float32),
        jax.ShapeDtypeStruct((T, LANES), jnp.float32),
        jax.ShapeDtypeStruct((1, 1), jnp.float32),
        jax.ShapeDtypeStruct((1, E), jnp.int32),
    )
    return pl.pallas_call(_gating_body, out_shape=out_shapes)(x3, wg)


def _dispatch_body(x_hbm, s1_hbm, s2_hbm, out_hbm, i1_v, i2_v, rows_v, sem):
    wid = lax.axis_index("s") * 2 + lax.axis_index("c")
    base = wid * TPW
    pltpu.sync_copy(s1_hbm.at[pl.ds(base, TPW)], i1_v)
    pltpu.sync_copy(s2_hbm.at[pl.ds(base, TPW)], i2_v)
    pltpu.sync_copy(x_hbm.at[pl.ds(base, TPW)], rows_v)
    pltpu.async_copy(rows_v, out_hbm.at[i1_v], sem).wait()
    pltpu.async_copy(rows_v, out_hbm.at[i2_v], sem).wait()


def _sc_mesh():
    return plsc.VectorSubcoreMesh(
        core_axis_name="c", subcore_axis_name="s", num_cores=2, num_subcores=16)


@functools.lru_cache(maxsize=None)
def _make_dispatch():
    return pl.kernel(
        _dispatch_body,
        out_type=jax.ShapeDtypeStruct(((E + 1) * C, D), jnp.float32),
        mesh=_sc_mesh(),
        scratch_types=[
            pltpu.VMEM((TPW,), jnp.int32),
            pltpu.VMEM((TPW,), jnp.int32),
            pltpu.VMEM((TPW, D), jnp.float32),
            pltpu.SemaphoreType.DMA,
        ],
    )


FB = 512


def _ffn_body(x_ref, w1_ref, b1_ref, w2_ref, b2_ref, out_ref):
    f = pl.program_id(1)
    w1b = w1_ref[0].astype(jnp.bfloat16)
    h = jnp.dot(x_ref[...].astype(jnp.bfloat16), w1b,
                preferred_element_type=jnp.float32)
    h = jnp.maximum(h + b1_ref[0], 0.0)
    w2b = w2_ref[0].astype(jnp.bfloat16)
    contrib = jnp.dot(h.astype(jnp.bfloat16), w2b,
                      preferred_element_type=jnp.float32)

    @pl.when(f == 0)
    def _init():
        out_ref[...] = contrib + b2_ref[0]

    @pl.when(f != 0)
    def _acc():
        out_ref[...] += contrib


def _ffn(disp, w1, b1, w2, b2):
    grid = (E, F // FB)
    return pl.pallas_call(
        _ffn_body,
        grid=grid,
        in_specs=[
            pl.BlockSpec((C, D), lambda e, f: (e, 0)),
            pl.BlockSpec((1, D, FB), lambda e, f: (e, 0, f)),
            pl.BlockSpec((1, 1, FB), lambda e, f: (e, 0, f)),
            pl.BlockSpec((1, FB, D), lambda e, f: (e, f, 0)),
            pl.BlockSpec((1, 1, D), lambda e, f: (e, 0, 0)),
        ],
        out_specs=pl.BlockSpec((C, D), lambda e, f: (e, 0)),
        out_shape=jax.ShapeDtypeStruct((E * C, D), jnp.float32),
    )(disp, w1, b1.reshape(E, 1, F), w2, b2.reshape(E, 1, D))


KC = 16
NCH = TPW // KC


def _combine_body(eo_hbm, s1_hbm, s2_hbm, w1_hbm, w2_hbm, out_hbm,
                  i1_v, i2_v, r1_v, r2_v, w1_v, w2_v, sem):
    wid = lax.axis_index("s") * 2 + lax.axis_index("c")
    base = wid * TPW

    def fire(ch):
        buf = ch % 2
        b = base + ch * KC
        pltpu.sync_copy(s1_hbm.at[pl.ds(b, KC)], i1_v.at[buf])
        pltpu.sync_copy(s2_hbm.at[pl.ds(b, KC)], i2_v.at[buf])
        pltpu.sync_copy(w1_hbm.at[pl.ds(b, KC)], w1_v.at[buf])
        pltpu.sync_copy(w2_hbm.at[pl.ds(b, KC)], w2_v.at[buf])
        c1 = pltpu.async_copy(eo_hbm.at[i1_v.at[buf]], r1_v.at[buf], sem)
        c2 = pltpu.async_copy(eo_hbm.at[i2_v.at[buf]], r2_v.at[buf], sem)
        return c1, c2

    pend = fire(0)
    for ch in range(NCH):
        buf = ch % 2
        cur = pend
        if ch + 1 < NCH:
            pend = fire(ch + 1)
        cur[0].wait()
        cur[1].wait()

        def token_body(t, carry):
            wa = w1_v[buf, t, :]
            wb = w2_v[buf, t, :]
            for j in range(D // LANES):
                sl = pl.ds(j * LANES, LANES)
                r1_v[buf, t, sl] = r1_v[buf, t, sl] * wa + r2_v[buf, t, sl] * wb
            return carry

        lax.fori_loop(0, KC, token_body, 0)
        pltpu.sync_copy(r1_v.at[buf], out_hbm.at[pl.ds(base + ch * KC, KC)])


@functools.lru_cache(maxsize=None)
def _make_combine():
    return pl.kernel(
        _combine_body,
        out_type=jax.ShapeDtypeStruct((T, D), jnp.float32),
        mesh=_sc_mesh(),
        scratch_types=[
            pltpu.VMEM((2, KC), jnp.int32),
            pltpu.VMEM((2, KC), jnp.int32),
            pltpu.VMEM((2, KC, D), jnp.float32),
            pltpu.VMEM((2, KC, D), jnp.float32),
            pltpu.VMEM((2, KC, LANES), jnp.float32),
            pltpu.VMEM((2, KC, LANES), jnp.float32),
            pltpu.SemaphoreType.DMA,
        ],
    )


def kernel(hidden_states, wg, w1, b1, w2, b2):
    S, B, _ = hidden_states.shape
    x2d = hidden_states.reshape(T, D)
    sd1, sd2, sc1, sc2, we1, we2, laux, counts = _gating(hidden_states, wg)
    disp = _make_dispatch()(x2d, sd1, sd2)
    eo = _ffn(disp, w1, b1, w2, b2)
    out = _make_combine()(eo, sc1, sc2, we1, we2)
    return (out.reshape(S, B, D), laux.reshape(()), counts.reshape(E))


# speedup vs baseline: 1.1051x; 1.0471x over previous
import functools

import jax
import jax.numpy as jnp
from jax import lax
from jax.experimental import pallas as pl
from jax.experimental.pallas import tpu as pltpu
from jax.experimental.pallas import tpu_sc as plsc

T = 2048
D = 1024
E = 8
F = 2048
C = 512
DUMMY = E * C
NW = 32
TPW = T // NW
LANES = 16


def _gating_body(x3_ref, wg_ref, sd1_ref, sd2_ref, sc1_ref, sc2_ref,
                 we1_ref, we2_ref, laux_ref, counts_ref):
    x = x3_ref[...].reshape(T, D)
    wg = wg_ref[...]
    logits = jnp.dot(x, wg, preferred_element_type=jnp.float32)
    lmax = jnp.max(logits, axis=-1, keepdims=True)
    ex = jnp.exp(logits - lmax)
    gates = ex / jnp.sum(ex, axis=-1, keepdims=True)

    col = lax.broadcasted_iota(jnp.int32, (T, E), 1)
    gmax = jnp.max(gates, axis=-1, keepdims=True)
    idx1 = jnp.min(jnp.where(gates == gmax, col, E), axis=-1, keepdims=True)
    mask1 = col == idx1
    neg_inf = jnp.float32(-jnp.inf)
    logits2 = jnp.where(mask1, neg_inf, logits)
    l2max = jnp.max(logits2, axis=-1, keepdims=True)
    idx2 = jnp.min(jnp.where(logits2 == l2max, col, E), axis=-1, keepdims=True)
    mask2 = col == idx2

    m1f = mask1.astype(jnp.float32)
    m2f = mask2.astype(jnp.float32)

    me = jnp.mean(gates, axis=0, keepdims=True)
    ce = jnp.mean(m1f, axis=0, keepdims=True)
    laux_ref[...] = jnp.sum(me * ce, axis=1, keepdims=True) * jnp.float32(E)

    mm = jnp.concatenate([m1f, m2f], axis=1)
    chunk = 512
    parts = []
    for c0 in range(0, T, chunk):
        ri = lax.broadcasted_iota(jnp.int32, (chunk, T), 0) + c0
        ci = lax.broadcasted_iota(jnp.int32, (chunk, T), 1)
        tri = (ci <= ri).astype(jnp.float32)
        parts.append(jnp.dot(tri, mm, preferred_element_type=jnp.float32))
    cs = jnp.concatenate(parts, axis=0)

    count1 = jnp.sum(m1f, axis=0, keepdims=True)
    loc1 = cs[:, :E] - 1.0
    loc2 = cs[:, E:] - 1.0 + count1

    cap = jnp.float32(C)
    m1c = m1f * (loc1 < cap).astype(jnp.float32)
    m2c = m2f * (loc2 < cap).astype(jnp.float32)

    counts_ref[0, :] = jnp.sum(m1c + m2c, axis=0).astype(jnp.int32)

    loc1s = jnp.sum(loc1 * m1c, axis=1, keepdims=True)
    loc2s = jnp.sum(loc2 * m2c, axis=1, keepdims=True)
    kept1 = jnp.sum(m1c, axis=1, keepdims=True)
    kept2 = jnp.sum(m2c, axis=1, keepdims=True)
    g1s = jnp.sum(gates * m1c, axis=1, keepdims=True)
    g2s = jnp.sum(gates * m2c, axis=1, keepdims=True)
    denom = jnp.maximum(g1s + g2s, 1e-9)
    we1_ref[...] = jnp.broadcast_to(g1s / denom, (T, LANES))
    we2_ref[...] = jnp.broadcast_to(g2s / denom, (T, LANES))

    base1 = idx1 * C + loc1s.astype(jnp.int32)
    base2 = idx2 * C + loc2s.astype(jnp.int32)
    sc1_ref[...] = base1.reshape(T)
    sc2_ref[...] = base2.reshape(T)
    sd1_ref[...] = jnp.where(kept1 > 0.5, base1, DUMMY).reshape(T)
    sd2_ref[...] = jnp.where(kept2 > 0.5, base2, DUMMY).reshape(T)


def _gating(x3, wg):
    out_shapes = (
        jax.ShapeDtypeStruct((T,), jnp.int32),
        jax.ShapeDtypeStruct((T,), jnp.int32),
        jax.ShapeDtypeStruct((T,), jnp.int32),
        jax.ShapeDtypeStruct((T,), jnp.int32),
        jax.ShapeDtypeStruct((T, LANES), jnp.float32),
        jax.ShapeDtypeStruct((T, LANES), jnp.float32),
        jax.ShapeDtypeStruct((1, 1), jnp.float32),
        jax.ShapeDtypeStruct((1, E), jnp.int32),
    )
    return pl.pallas_call(_gating_body, out_shape=out_shapes)(x3, wg)


def _dispatch_body(x_hbm, s1_hbm, s2_hbm, out_hbm, i1_v, i2_v, rows_v, sem):
    wid = lax.axis_index("s") * 2 + lax.axis_index("c")
    base = wid * TPW
    pltpu.sync_copy(s1_hbm.at[pl.ds(base, TPW)], i1_v)
    pltpu.sync_copy(s2_hbm.at[pl.ds(base, TPW)], i2_v)
    pltpu.sync_copy(x_hbm.at[pl.ds(base, TPW)], rows_v)
    pltpu.async_copy(rows_v, out_hbm.at[i1_v], sem).wait()
    pltpu.async_copy(rows_v, out_hbm.at[i2_v], sem).wait()


def _sc_mesh():
    return plsc.VectorSubcoreMesh(
        core_axis_name="c", subcore_axis_name="s", num_cores=2, num_subcores=16)


@functools.lru_cache(maxsize=None)
def _make_dispatch():
    return pl.kernel(
        _dispatch_body,
        out_type=jax.ShapeDtypeStruct(((E + 1) * C, D), jnp.float32),
        mesh=_sc_mesh(),
        scratch_types=[
            pltpu.VMEM((TPW,), jnp.int32),
            pltpu.VMEM((TPW,), jnp.int32),
            pltpu.VMEM((TPW, D), jnp.float32),
            pltpu.SemaphoreType.DMA,
        ],
    )


FB = 512


def _ffn_body(x_ref, w1_ref, b1_ref, w2_ref, b2_ref, out_ref):
    f = pl.program_id(1)
    w1b = w1_ref[0].astype(jnp.bfloat16)
    h = jnp.dot(x_ref[...].astype(jnp.bfloat16), w1b,
                preferred_element_type=jnp.float32)
    h = jnp.maximum(h + b1_ref[0], 0.0)
    w2b = w2_ref[0].astype(jnp.bfloat16)
    contrib = jnp.dot(h.astype(jnp.bfloat16), w2b,
                      preferred_element_type=jnp.float32)

    @pl.when(f == 0)
    def _init():
        out_ref[...] = contrib + b2_ref[0]

    @pl.when(f != 0)
    def _acc():
        out_ref[...] += contrib


def _ffn(disp, w1, b1, w2, b2):
    grid = (E, F // FB)
    return pl.pallas_call(
        _ffn_body,
        grid=grid,
        in_specs=[
            pl.BlockSpec((C, D), lambda e, f: (e, 0)),
            pl.BlockSpec((1, D, FB), lambda e, f: (e, 0, f)),
            pl.BlockSpec((1, 1, FB), lambda e, f: (e, 0, f)),
            pl.BlockSpec((1, FB, D), lambda e, f: (e, f, 0)),
            pl.BlockSpec((1, 1, D), lambda e, f: (e, 0, 0)),
        ],
        out_specs=pl.BlockSpec((C, D), lambda e, f: (e, 0)),
        out_shape=jax.ShapeDtypeStruct((E * C, D), jnp.float32),
    )(disp, w1, b1.reshape(E, 1, F), w2, b2.reshape(E, 1, D))


KC = 16
NCH = TPW // KC


def _combine_body(eo_hbm, s1_hbm, s2_hbm, w1_hbm, w2_hbm, out_hbm,
                  i1_v, i2_v, r1_v, r2_v, w1_v, w2_v, sem):
    wid = lax.axis_index("s") * 2 + lax.axis_index("c")
    base = wid * TPW

    def fire(ch):
        buf = ch % 2
        b = base + ch * KC
        pltpu.sync_copy(s1_hbm.at[pl.ds(b, KC)], i1_v.at[buf])
        pltpu.sync_copy(s2_hbm.at[pl.ds(b, KC)], i2_v.at[buf])
        pltpu.sync_copy(w1_hbm.at[pl.ds(b, KC)], w1_v.at[buf])
        pltpu.sync_copy(w2_hbm.at[pl.ds(b, KC)], w2_v.at[buf])
        c1 = pltpu.async_copy(eo_hbm.at[i1_v.at[buf]], r1_v.at[buf], sem)
        c2 = pltpu.async_copy(eo_hbm.at[i2_v.at[buf]], r2_v.at[buf], sem)
        return c1, c2

    pend = fire(0)
    for ch in range(NCH):
        buf = ch % 2
        cur = pend
        if ch + 1 < NCH:
            pend = fire(ch + 1)
        cur[0].wait()
        cur[1].wait()

        def group_body(g):
            t = g // (D // LANES)
            j = g % (D // LANES)
            wa = w1_v[buf, t, :]
            wb = w2_v[buf, t, :]
            sl = pl.ds(j * LANES, LANES)
            r1_v[buf, t, sl] = r1_v[buf, t, sl] * wa + r2_v[buf, t, sl] * wb

        plsc.parallel_loop(0, KC * (D // LANES), 1, unroll=8)(group_body)
        pltpu.sync_copy(r1_v.at[buf], out_hbm.at[pl.ds(base + ch * KC, KC)])


@functools.lru_cache(maxsize=None)
def _make_combine():
    return pl.kernel(
        _combine_body,
        out_type=jax.ShapeDtypeStruct((T, D), jnp.float32),
        mesh=_sc_mesh(),
        scratch_types=[
            pltpu.VMEM((2, KC), jnp.int32),
            pltpu.VMEM((2, KC), jnp.int32),
            pltpu.VMEM((2, KC, D), jnp.float32),
            pltpu.VMEM((2, KC, D), jnp.float32),
            pltpu.VMEM((2, KC, LANES), jnp.float32),
            pltpu.VMEM((2, KC, LANES), jnp.float32),
            pltpu.SemaphoreType.DMA,
        ],
    )


def kernel(hidden_states, wg, w1, b1, w2, b2):
    S, B, _ = hidden_states.shape
    x2d = hidden_states.reshape(T, D)
    sd1, sd2, sc1, sc2, we1, we2, laux, counts = _gating(hidden_states, wg)
    disp = _make_dispatch()(x2d, sd1, sd2)
    eo = _ffn(disp, w1, b1, w2, b2)
    out = _make_combine()(eo, sc1, sc2, we1, we2)
    return (out.reshape(S, B, D), laux.reshape(()), counts.reshape(E))


# speedup vs baseline: 1.1086x; 1.0031x over previous
import functools

import jax
import jax.numpy as jnp
from jax import lax
from jax.experimental import pallas as pl
from jax.experimental.pallas import tpu as pltpu
from jax.experimental.pallas import tpu_sc as plsc

T = 2048
D = 1024
E = 8
F = 2048
C = 512
DUMMY = E * C
NW = 32
TPW = T // NW
LANES = 16


def _gating_body(x3_ref, wg_ref, sd1_ref, sd2_ref, sc1_ref, sc2_ref,
                 we1_ref, we2_ref, laux_ref, counts_ref):
    x = x3_ref[...].reshape(T, D)
    wg = wg_ref[...]
    logits = jnp.dot(x, wg, preferred_element_type=jnp.float32)
    lmax = jnp.max(logits, axis=-1, keepdims=True)
    ex = jnp.exp(logits - lmax)
    gates = ex / jnp.sum(ex, axis=-1, keepdims=True)

    col = lax.broadcasted_iota(jnp.int32, (T, E), 1)
    gmax = jnp.max(gates, axis=-1, keepdims=True)
    idx1 = jnp.min(jnp.where(gates == gmax, col, E), axis=-1, keepdims=True)
    mask1 = col == idx1
    neg_inf = jnp.float32(-jnp.inf)
    logits2 = jnp.where(mask1, neg_inf, logits)
    l2max = jnp.max(logits2, axis=-1, keepdims=True)
    idx2 = jnp.min(jnp.where(logits2 == l2max, col, E), axis=-1, keepdims=True)
    mask2 = col == idx2

    m1f = mask1.astype(jnp.float32)
    m2f = mask2.astype(jnp.float32)

    me = jnp.mean(gates, axis=0, keepdims=True)
    ce = jnp.mean(m1f, axis=0, keepdims=True)
    laux_ref[...] = jnp.sum(me * ce, axis=1, keepdims=True) * jnp.float32(E)

    mm = jnp.concatenate([m1f, m2f], axis=1).astype(jnp.bfloat16)
    chunk = 512
    parts = []
    for c0 in range(0, T, chunk):
        ri = lax.broadcasted_iota(jnp.int32, (chunk, T), 0) + c0
        ci = lax.broadcasted_iota(jnp.int32, (chunk, T), 1)
        tri = (ci <= ri).astype(jnp.bfloat16)
        parts.append(jnp.dot(tri, mm, preferred_element_type=jnp.float32))
    cs = jnp.concatenate(parts, axis=0)

    count1 = jnp.sum(m1f, axis=0, keepdims=True)
    loc1 = cs[:, :E] - 1.0
    loc2 = cs[:, E:] - 1.0 + count1

    cap = jnp.float32(C)
    m1c = m1f * (loc1 < cap).astype(jnp.float32)
    m2c = m2f * (loc2 < cap).astype(jnp.float32)

    counts_ref[0, :] = jnp.sum(m1c + m2c, axis=0).astype(jnp.int32)

    loc1s = jnp.sum(loc1 * m1c, axis=1, keepdims=True)
    loc2s = jnp.sum(loc2 * m2c, axis=1, keepdims=True)
    kept1 = jnp.sum(m1c, axis=1, keepdims=True)
    kept2 = jnp.sum(m2c, axis=1, keepdims=True)
    g1s = jnp.sum(gates * m1c, axis=1, keepdims=True)
    g2s = jnp.sum(gates * m2c, axis=1, keepdims=True)
    denom = jnp.maximum(g1s + g2s, 1e-9)
    we1_ref[...] = jnp.broadcast_to(g1s / denom, (T, LANES))
    we2_ref[...] = jnp.broadcast_to(g2s / denom, (T, LANES))

    base1 = idx1 * C + loc1s.astype(jnp.int32)
    base2 = idx2 * C + loc2s.astype(jnp.int32)
    sc1_ref[...] = base1.reshape(T)
    sc2_ref[...] = base2.reshape(T)
    sd1_ref[...] = jnp.where(kept1 > 0.5, base1, DUMMY).reshape(T)
    sd2_ref[...] = jnp.where(kept2 > 0.5, base2, DUMMY).reshape(T)


def _gating(x3, wg):
    out_shapes = (
        jax.ShapeDtypeStruct((T,), jnp.int32),
        jax.ShapeDtypeStruct((T,), jnp.int32),
        jax.ShapeDtypeStruct((T,), jnp.int32),
        jax.ShapeDtypeStruct((T,), jnp.int32),
        jax.ShapeDtypeStruct((T, LANES), jnp.float32),
        jax.ShapeDtypeStruct((T, LANES), jnp.float32),
        jax.ShapeDtypeStruct((1, 1), jnp.float32),
        jax.ShapeDtypeStruct((1, E), jnp.int32),
    )
    return pl.pallas_call(_gating_body, out_shape=out_shapes)(x3, wg)


def _dispatch_body(x_hbm, s1_hbm, s2_hbm, out_hbm, i1_v, i2_v, rows_v, sem):
    wid = lax.axis_index("s") * 2 + lax.axis_index("c")
    base = wid * TPW
    pltpu.sync_copy(s1_hbm.at[pl.ds(base, TPW)], i1_v)
    pltpu.sync_copy(s2_hbm.at[pl.ds(base, TPW)], i2_v)
    pltpu.sync_copy(x_hbm.at[pl.ds(base, TPW)], rows_v)
    pltpu.async_copy(rows_v, out_hbm.at[i1_v], sem).wait()
    pltpu.async_copy(rows_v, out_hbm.at[i2_v], sem).wait()


def _sc_mesh():
    return plsc.VectorSubcoreMesh(
        core_axis_name="c", subcore_axis_name="s", num_cores=2, num_subcores=16)


@functools.lru_cache(maxsize=None)
def _make_dispatch():
    return pl.kernel(
        _dispatch_body,
        out_type=jax.ShapeDtypeStruct(((E + 1) * C, D), jnp.float32),
        mesh=_sc_mesh(),
        scratch_types=[
            pltpu.VMEM((TPW,), jnp.int32),
            pltpu.VMEM((TPW,), jnp.int32),
            pltpu.VMEM((TPW, D), jnp.float32),
            pltpu.SemaphoreType.DMA,
        ],
    )


FB = 512


def _ffn_body(x_ref, w1_ref, b1_ref, w2_ref, b2_ref, out_ref):
    f = pl.program_id(1)
    w1b = w1_ref[0].astype(jnp.bfloat16)
    h = jnp.dot(x_ref[...].astype(jnp.bfloat16), w1b,
                preferred_element_type=jnp.float32)
    h = jnp.maximum(h + b1_ref[0], 0.0)
    w2b = w2_ref[0].astype(jnp.bfloat16)
    contrib = jnp.dot(h.astype(jnp.bfloat16), w2b,
                      preferred_element_type=jnp.float32)

    @pl.when(f == 0)
    def _init():
        out_ref[...] = contrib + b2_ref[0]

    @pl.when(f != 0)
    def _acc():
        out_ref[...] += contrib


def _ffn(disp, w1, b1, w2, b2):
    grid = (E, F // FB)
    return pl.pallas_call(
        _ffn_body,
        grid=grid,
        in_specs=[
            pl.BlockSpec((C, D), lambda e, f: (e, 0)),
            pl.BlockSpec((1, D, FB), lambda e, f: (e, 0, f)),
            pl.BlockSpec((1, 1, FB), lambda e, f: (e, 0, f)),
            pl.BlockSpec((1, FB, D), lambda e, f: (e, f, 0)),
            pl.BlockSpec((1, 1, D), lambda e, f: (e, 0, 0)),
        ],
        out_specs=pl.BlockSpec((C, D), lambda e, f: (e, 0)),
        out_shape=jax.ShapeDtypeStruct((E * C, D), jnp.float32),
    )(disp, w1, b1.reshape(E, 1, F), w2, b2.reshape(E, 1, D))


KC = 16
NCH = TPW // KC


def _combine_body(eo_hbm, s1_hbm, s2_hbm, w1_hbm, w2_hbm, out_hbm,
                  i1_v, i2_v, r1_v, r2_v, w1_v, w2_v, sem):
    wid = lax.axis_index("s") * 2 + lax.axis_index("c")
    base = wid * TPW

    def fire(ch):
        buf = ch % 2
        b = base + ch * KC
        pltpu.sync_copy(s1_hbm.at[pl.ds(b, KC)], i1_v.at[buf])
        pltpu.sync_copy(s2_hbm.at[pl.ds(b, KC)], i2_v.at[buf])
        pltpu.sync_copy(w1_hbm.at[pl.ds(b, KC)], w1_v.at[buf])
        pltpu.sync_copy(w2_hbm.at[pl.ds(b, KC)], w2_v.at[buf])
        c1 = pltpu.async_copy(eo_hbm.at[i1_v.at[buf]], r1_v.at[buf], sem)
        c2 = pltpu.async_copy(eo_hbm.at[i2_v.at[buf]], r2_v.at[buf], sem)
        return c1, c2

    pend = fire(0)
    for ch in range(NCH):
        buf = ch % 2
        cur = pend
        if ch + 1 < NCH:
            pend = fire(ch + 1)
        cur[0].wait()
        cur[1].wait()

        def group_body(g):
            t = g // (D // LANES)
            j = g % (D // LANES)
            wa = w1_v[buf, t, :]
            wb = w2_v[buf, t, :]
            sl = pl.ds(j * LANES, LANES)
            r1_v[buf, t, sl] = r1_v[buf, t, sl] * wa + r2_v[buf, t, sl] * wb

        plsc.parallel_loop(0, KC * (D // LANES), 1, unroll=8)(group_body)
        pltpu.sync_copy(r1_v.at[buf], out_hbm.at[pl.ds(base + ch * KC, KC)])


@functools.lru_cache(maxsize=None)
def _make_combine():
    return pl.kernel(
        _combine_body,
        out_type=jax.ShapeDtypeStruct((T, D), jnp.float32),
        mesh=_sc_mesh(),
        scratch_types=[
            pltpu.VMEM((2, KC), jnp.int32),
            pltpu.VMEM((2, KC), jnp.int32),
            pltpu.VMEM((2, KC, D), jnp.float32),
            pltpu.VMEM((2, KC, D), jnp.float32),
            pltpu.VMEM((2, KC, LANES), jnp.float32),
            pltpu.VMEM((2, KC, LANES), jnp.float32),
            pltpu.SemaphoreType.DMA,
        ],
    )


def kernel(hidden_states, wg, w1, b1, w2, b2):
    S, B, _ = hidden_states.shape
    x2d = hidden_states.reshape(T, D)
    sd1, sd2, sc1, sc2, we1, we2, laux, counts = _gating(hidden_states, wg)
    disp = _make_dispatch()(x2d, sd1, sd2)
    eo = _ffn(disp, w1, b1, w2, b2)
    out = _make_combine()(eo, sc1, sc2, we1, we2)
    return (out.reshape(S, B, D), laux.reshape(()), counts.reshape(E))
